# baseline (speedup 1.0000x reference)
import jax
import jax.numpy as jnp
from jax.experimental import pallas as pl
from jax.experimental.pallas import tpu as pltpu

_HID = 128
_NMP = 6
_NLAYERS = 2


def _fused_body(f_ref, wt_ref, bt_ref, wenc_ref, benc_ref, wv_ref, sb_ref,
                ones_ref, sel_ref, wc_ref, bc_ref, out_ref):
    f = f_ref[...].astype(jnp.bfloat16)
    h = jnp.dot(f, wt_ref[0], preferred_element_type=jnp.float32) + bt_ref[0]
    for l in range(_NLAYERS):
        hb = h.astype(jnp.bfloat16)
        outs = jnp.dot(hb, wenc_ref[l],
                       preferred_element_type=jnp.float32) + benc_ref[l]
        s = jnp.dot(hb, wv_ref[l],
                    preferred_element_type=jnp.float32) + sb_ref[l]
        s = jnp.where(s >= 0, s, 0.2 * s)
        e = jnp.exp(s)
        ebf = e.astype(jnp.bfloat16)
        denom = jnp.dot(ebf, ones_ref[...],
                        preferred_element_type=jnp.float32)
        eb = jnp.dot(ebf, sel_ref[...],
                     preferred_element_type=jnp.float32)
        p = eb * outs
        acc = ((p[:, 0:_HID] + p[:, _HID:2 * _HID])
               + (p[:, 2 * _HID:3 * _HID] + p[:, 3 * _HID:4 * _HID])
               + (p[:, 4 * _HID:5 * _HID] + p[:, 5 * _HID:6 * _HID]))
        acc = acc / denom
        h = jnp.where(acc > 0, acc, jnp.exp(jnp.minimum(acc, 0.0)) - 1.0)
    out_ref[...] = jnp.dot(h.astype(jnp.bfloat16), wc_ref[...],
                           preferred_element_type=jnp.float32) + bc_ref[0]


def kernel(x, edge_index, feat_author, feat_paper, feat_term, feat_conf,
           Wt, bt, Wenc, benc, Watt, batt, Wc, bc):
    del x, edge_index
    feats = [feat_author, feat_paper, feat_term, feat_conf]
    d_in = feats[0].shape[1]
    n_cls = Wc.shape[1]

    Wenc2 = jnp.transpose(Wenc, (0, 2, 1, 3)).reshape(_NLAYERS, _HID, _NMP * _HID)
    benc2 = benc.reshape(_NLAYERS, _NMP * _HID)
    WV = jnp.einsum('lmdk,lk->ldm', Wenc, Watt)
    WV = jnp.pad(WV, ((0, 0), (0, 0), (0, _HID - _NMP)))
    sb = jnp.einsum('lmk,lk->lm', benc, Watt) + batt[:, None]
    sb = jnp.pad(sb, ((0, 0), (0, _HID - _NMP)), constant_values=-1e9)
    ones_m = jnp.ones((_HID, _HID), jnp.bfloat16)
    lane = jnp.arange(_NMP * _HID) // _HID
    sel = (lane[None, :] == jnp.arange(_HID)[:, None]).astype(jnp.bfloat16)
    bc2 = bc.reshape(1, n_cls)
    bt3 = bt.reshape(4, 1, _HID)

    def _run_type(f, t, blk):
        n = f.shape[0]
        nb = n // blk
        return pl.pallas_call(
            _fused_body,
            grid=(nb,),
            in_specs=[
                pl.BlockSpec((blk, d_in), lambda i: (i, 0)),
                pl.BlockSpec((1, d_in, _HID), lambda i, _t=t: (_t, 0, 0)),
                pl.BlockSpec((1, 1, _HID), lambda i, _t=t: (_t, 0, 0)),
                pl.BlockSpec((_NLAYERS, _HID, _NMP * _HID), lambda i: (0, 0, 0)),
                pl.BlockSpec((_NLAYERS, _NMP * _HID), lambda i: (0, 0)),
                pl.BlockSpec((_NLAYERS, _HID, _HID), lambda i: (0, 0, 0)),
                pl.BlockSpec((_NLAYERS, _HID), lambda i: (0, 0)),
                pl.BlockSpec((_HID, _HID), lambda i: (0, 0)),
                pl.BlockSpec((_HID, _NMP * _HID), lambda i: (0, 0)),
                pl.BlockSpec((_HID, n_cls), lambda i: (0, 0)),
                pl.BlockSpec((1, n_cls), lambda i: (0, 0)),
            ],
            out_specs=pl.BlockSpec((blk, n_cls), lambda i: (i, 0)),
            out_shape=jax.ShapeDtypeStruct((n, n_cls), jnp.float32),
            compiler_params=pltpu.CompilerParams(
                dimension_semantics=("arbitrary",)),
        )(f, Wt.astype(jnp.bfloat16), bt3, Wenc2.astype(jnp.bfloat16), benc2, WV.astype(jnp.bfloat16), sb, ones_m, sel, Wc.astype(jnp.bfloat16), bc2)

    outs = [
        _run_type(feat_author, 0, 1000),
        _run_type(feat_paper, 1, 1000),
        _run_type(feat_term, 2, feat_term.shape[0]),
        _run_type(feat_conf, 3, feat_conf.shape[0]),
    ]
    return jnp.concatenate(outs, axis=0)


# speedup vs baseline: 1.0507x; 1.0507x over previous
import jax
import jax.numpy as jnp
from jax.experimental import pallas as pl
from jax.experimental.pallas import tpu as pltpu

_HID = 128
_NMP = 6
_NLAYERS = 2
_BLK = 1000
_NB = (4, 4, 2, 1)
_B0, _B1, _B2 = _NB[0], _NB[0] + _NB[1], _NB[0] + _NB[1] + _NB[2]


def _type_of(i):
    return jnp.where(i < _B0, 0, jnp.where(i < _B1, 1, jnp.where(i < _B2, 2, 3)))


def _fused_body(fa_ref, fp_ref, ft_ref, fc_ref, wt_ref, bt_ref, wenc_ref,
                benc_ref, wv_ref, sb_ref, ones_ref, sel_ref, wc_ref, bc_ref,
                out_ref):
    i = pl.program_id(0)
    f = jnp.where(i < _B0, fa_ref[...],
                  jnp.where(i < _B1, fp_ref[...],
                            jnp.where(i < _B2, ft_ref[...], fc_ref[...])))
    h = jnp.dot(f, wt_ref[0], preferred_element_type=jnp.float32) + bt_ref[0]
    for l in range(_NLAYERS):
        outs = jnp.dot(h, wenc_ref[l],
                       preferred_element_type=jnp.float32) + benc_ref[l]
        s = jnp.dot(h, wv_ref[l],
                    preferred_element_type=jnp.float32) + sb_ref[l]
        s = jnp.where(s >= 0, s, 0.2 * s)
        e = jnp.exp(s)
        denom = jnp.dot(e, ones_ref[...],
                        preferred_element_type=jnp.float32)
        eb = jnp.dot(e, sel_ref[...],
                     preferred_element_type=jnp.float32)
        p = eb * outs
        acc = ((p[:, 0:_HID] + p[:, _HID:2 * _HID])
               + (p[:, 2 * _HID:3 * _HID] + p[:, 3 * _HID:4 * _HID])
               + (p[:, 4 * _HID:5 * _HID] + p[:, 5 * _HID:6 * _HID]))
        acc = acc / denom
        h = jnp.where(acc > 0, acc, jnp.exp(jnp.minimum(acc, 0.0)) - 1.0)
    out_ref[...] = jnp.dot(h, wc_ref[...],
                           preferred_element_type=jnp.float32) + bc_ref[0]


def kernel(x, edge_index, feat_author, feat_paper, feat_term, feat_conf,
           Wt, bt, Wenc, benc, Watt, batt, Wc, bc):
    del x, edge_index
    d_in = feat_author.shape[1]
    n_cls = Wc.shape[1]
    n_a, n_p, n_t, n_c = (feat_author.shape[0], feat_paper.shape[0],
                          feat_term.shape[0], feat_conf.shape[0])

    ft_pad = jnp.pad(feat_term, ((0, _NB[2] * _BLK - n_t), (0, 0)))
    fc_pad = jnp.pad(feat_conf, ((0, _NB[3] * _BLK - n_c), (0, 0)))

    Wenc2 = jnp.transpose(Wenc, (0, 2, 1, 3)).reshape(_NLAYERS, _HID, _NMP * _HID)
    benc2 = benc.reshape(_NLAYERS, _NMP * _HID)
    WV = jnp.einsum('lmdk,lk->ldm', Wenc, Watt)
    WV = jnp.pad(WV, ((0, 0), (0, 0), (0, _HID - _NMP)))
    sb = jnp.einsum('lmk,lk->lm', benc, Watt) + batt[:, None]
    sb = jnp.pad(sb, ((0, 0), (0, _HID - _NMP)), constant_values=-1e9)
    ones_m = jnp.ones((_HID, _HID), jnp.float32)
    lane = jnp.arange(_NMP * _HID) // _HID
    sel = (lane[None, :] == jnp.arange(_HID)[:, None]).astype(jnp.float32)
    bc2 = bc.reshape(1, n_cls)
    bt3 = bt.reshape(4, 1, _HID)

    n_blocks = sum(_NB)
    out = pl.pallas_call(
        _fused_body,
        grid=(n_blocks,),
        in_specs=[
            pl.BlockSpec((_BLK, d_in), lambda i: (jnp.clip(i, 0, _NB[0] - 1), 0)),
            pl.BlockSpec((_BLK, d_in), lambda i: (jnp.clip(i - _B0, 0, _NB[1] - 1), 0)),
            pl.BlockSpec((_BLK, d_in), lambda i: (jnp.clip(i - _B1, 0, _NB[2] - 1), 0)),
            pl.BlockSpec((_BLK, d_in), lambda i: (jnp.clip(i - _B2, 0, _NB[3] - 1), 0)),
            pl.BlockSpec((1, d_in, _HID), lambda i: (_type_of(i), 0, 0)),
            pl.BlockSpec((1, 1, _HID), lambda i: (_type_of(i), 0, 0)),
            pl.BlockSpec((_NLAYERS, _HID, _NMP * _HID), lambda i: (0, 0, 0)),
            pl.BlockSpec((_NLAYERS, _NMP * _HID), lambda i: (0, 0)),
            pl.BlockSpec((_NLAYERS, _HID, _HID), lambda i: (0, 0, 0)),
            pl.BlockSpec((_NLAYERS, _HID), lambda i: (0, 0)),
            pl.BlockSpec((_HID, _HID), lambda i: (0, 0)),
            pl.BlockSpec((_HID, _NMP * _HID), lambda i: (0, 0)),
            pl.BlockSpec((_HID, n_cls), lambda i: (0, 0)),
            pl.BlockSpec((1, n_cls), lambda i: (0, 0)),
        ],
        out_specs=pl.BlockSpec((_BLK, n_cls), lambda i: (i, 0)),
        out_shape=jax.ShapeDtypeStruct((n_blocks * _BLK, n_cls), jnp.float32),
        compiler_params=pltpu.CompilerParams(
            dimension_semantics=("arbitrary",)),
    )(feat_author, feat_paper, ft_pad, fc_pad, Wt, bt3, Wenc2, benc2, WV, sb,
      ones_m, sel, Wc, bc2)

    return jnp.concatenate(
        [out[0:n_a + n_p], out[_B1 * _BLK:_B1 * _BLK + n_t],
         out[_B2 * _BLK:_B2 * _BLK + n_c]], axis=0)


# speedup vs baseline: 1.1116x; 1.0579x over previous
import jax
import jax.numpy as jnp
from jax.experimental import pallas as pl
from jax.experimental.pallas import tpu as pltpu

_HID = 128
_NMP = 6
_NLAYERS = 2
_BLK = 1000
_B0, _B1, _B2 = 4, 8, 9
_NBLK = 10


def _fused_body(fa_ref, fp_ref, ft_ref, fc_ref, wt_ref, bt_ref, wenc_ref,
                benc_ref, wv_ref, sb_ref, ones_ref, sel_ref, wc_ref, bc_ref,
                out_ref):
    i = pl.program_id(0)
    t = jnp.where(i < _B0, 0, jnp.where(i < _B1, 1, 2))
    row = jax.lax.broadcasted_iota(jnp.int32, (_BLK, _HID), 0)
    primary = (i < _B2) | (row < 500)
    f = jnp.where(i < _B0, fa_ref[...],
                  jnp.where(i < _B1, fp_ref[...],
                            jnp.where(primary, ft_ref[...], fc_ref[...])))
    h0 = jnp.dot(f, wt_ref[t], preferred_element_type=jnp.float32) + bt_ref[t, 0]
    h1 = jnp.dot(f, wt_ref[3], preferred_element_type=jnp.float32) + bt_ref[3, 0]
    h = jnp.where(primary, h0, h1)
    for l in range(_NLAYERS):
        outs = jnp.dot(h, wenc_ref[l],
                       preferred_element_type=jnp.float32) + benc_ref[l]
        s = jnp.dot(h, wv_ref[l],
                    preferred_element_type=jnp.float32) + sb_ref[l]
        s = jnp.where(s >= 0, s, 0.2 * s)
        e = jnp.exp(s)
        denom = jnp.dot(e, ones_ref[...],
                        preferred_element_type=jnp.float32)
        eb = jnp.dot(e, sel_ref[...],
                     preferred_element_type=jnp.float32)
        p = eb * outs
        acc = ((p[:, 0:_HID] + p[:, _HID:2 * _HID])
               + (p[:, 2 * _HID:3 * _HID] + p[:, 3 * _HID:4 * _HID])
               + (p[:, 4 * _HID:5 * _HID] + p[:, 5 * _HID:6 * _HID]))
        acc = acc / denom
        h = jnp.where(acc > 0, acc, jnp.exp(jnp.minimum(acc, 0.0)) - 1.0)
    out_ref[...] = jnp.dot(h, wc_ref[...],
                           preferred_element_type=jnp.float32) + bc_ref[0]


def kernel(x, edge_index, feat_author, feat_paper, feat_term, feat_conf,
           Wt, bt, Wenc, benc, Watt, batt, Wc, bc):
    del x, edge_index
    d_in = feat_author.shape[1]
    n_cls = Wc.shape[1]
    n_t, n_c = feat_term.shape[0], feat_conf.shape[0]

    ft_pad = jnp.pad(feat_term, ((0, 2 * _BLK - n_t), (0, 0)))
    fc_pad = jnp.pad(feat_conf, ((_BLK - n_c, 0), (0, 0)))

    Wenc2 = jnp.transpose(Wenc, (0, 2, 1, 3)).reshape(_NLAYERS, _HID, _NMP * _HID)
    benc2 = benc.reshape(_NLAYERS, _NMP * _HID)
    WV = jnp.einsum('lmdk,lk->ldm', Wenc, Watt)
    WV = jnp.pad(WV, ((0, 0), (0, 0), (0, _HID - _NMP)))
    sb = jnp.einsum('lmk,lk->lm', benc, Watt) + batt[:, None]
    sb = jnp.pad(sb, ((0, 0), (0, _HID - _NMP)), constant_values=-1e9)
    ones_m = jnp.ones((_HID, _HID), jnp.float32)
    lane = jnp.arange(_NMP * _HID) // _HID
    sel = (lane[None, :] == jnp.arange(_HID)[:, None]).astype(jnp.float32)
    bc2 = bc.reshape(1, n_cls)
    bt3 = bt.reshape(4, 1, _HID)

    return pl.pallas_call(
        _fused_body,
        grid=(_NBLK,),
        in_specs=[
            pl.BlockSpec((_BLK, d_in), lambda i: (jnp.clip(i, 0, 3), 0)),
            pl.BlockSpec((_BLK, d_in), lambda i: (jnp.clip(i - _B0, 0, 3), 0)),
            pl.BlockSpec((_BLK, d_in), lambda i: (jnp.clip(i - _B1, 0, 1), 0)),
            pl.BlockSpec((_BLK, d_in), lambda i: (0, 0)),
            pl.BlockSpec((4, d_in, _HID), lambda i: (0, 0, 0)),
            pl.BlockSpec((4, 1, _HID), lambda i: (0, 0, 0)),
            pl.BlockSpec((_NLAYERS, _HID, _NMP * _HID), lambda i: (0, 0, 0)),
            pl.BlockSpec((_NLAYERS, _NMP * _HID), lambda i: (0, 0)),
            pl.BlockSpec((_NLAYERS, _HID, _HID), lambda i: (0, 0, 0)),
            pl.BlockSpec((_NLAYERS, _HID), lambda i: (0, 0)),
            pl.BlockSpec((_HID, _HID), lambda i: (0, 0)),
            pl.BlockSpec((_HID, _NMP * _HID), lambda i: (0, 0)),
            pl.BlockSpec((_HID, n_cls), lambda i: (0, 0)),
            pl.BlockSpec((1, n_cls), lambda i: (0, 0)),
        ],
        out_specs=pl.BlockSpec((_BLK, n_cls), lambda i: (i, 0)),
        out_shape=jax.ShapeDtypeStruct((_NBLK * _BLK, n_cls), jnp.float32),
        compiler_params=pltpu.CompilerParams(
            dimension_semantics=("arbitrary",)),
    )(feat_author, feat_paper, ft_pad, fc_pad, Wt, bt3, Wenc2, benc2, WV, sb,
      ones_m, sel, Wc, bc2)


# speedup vs baseline: 1.2766x; 1.1485x over previous
import jax
import jax.numpy as jnp
from jax.experimental import pallas as pl
from jax.experimental.pallas import tpu as pltpu

_HID = 128
_NMP = 6
_NLAYERS = 2
_BLK = 2000
_B0, _B1, _B2 = 2, 4, 4
_NBLK = 5
_TERM_ROWS = 1500


def _fused_body(fa_ref, fp_ref, ft_ref, fc_ref, wt_ref, bt_ref, wenc_ref,
                benc_ref, wv_ref, sb_ref, ones_ref, sel_ref, wc_ref, bc_ref,
                out_ref):
    i = pl.program_id(0)
    t = jnp.where(i < _B0, 0, jnp.where(i < _B1, 1, 2))
    row = jax.lax.broadcasted_iota(jnp.int32, (_BLK, _HID), 0)
    primary = (i < _B2) | (row < _TERM_ROWS)
    f = jnp.where(i < _B0, fa_ref[...],
                  jnp.where(i < _B1, fp_ref[...],
                            jnp.where(primary, ft_ref[...], fc_ref[...])))
    h0 = jnp.dot(f, wt_ref[t], preferred_element_type=jnp.float32) + bt_ref[t, 0]
    h1 = jnp.dot(f, wt_ref[3], preferred_element_type=jnp.float32) + bt_ref[3, 0]
    h = jnp.where(primary, h0, h1)
    for l in range(_NLAYERS):
        outs = jnp.dot(h, wenc_ref[l],
                       preferred_element_type=jnp.float32) + benc_ref[l]
        s = jnp.dot(h, wv_ref[l],
                    preferred_element_type=jnp.float32) + sb_ref[l]
        s = jnp.where(s >= 0, s, 0.2 * s)
        e = jnp.exp(s)
        denom = jnp.dot(e, ones_ref[...],
                        preferred_element_type=jnp.float32)
        eb = jnp.dot(e, sel_ref[...],
                     preferred_element_type=jnp.float32)
        p = eb * outs
        acc = ((p[:, 0:_HID] + p[:, _HID:2 * _HID])
               + (p[:, 2 * _HID:3 * _HID] + p[:, 3 * _HID:4 * _HID])
               + (p[:, 4 * _HID:5 * _HID] + p[:, 5 * _HID:6 * _HID]))
        acc = acc / denom
        h = jnp.where(acc > 0, acc, jnp.exp(jnp.minimum(acc, 0.0)) - 1.0)
    out_ref[...] = jnp.dot(h, wc_ref[...],
                           preferred_element_type=jnp.float32) + bc_ref[0]


def kernel(x, edge_index, feat_author, feat_paper, feat_term, feat_conf,
           Wt, bt, Wenc, benc, Watt, batt, Wc, bc):
    del x, edge_index
    d_in = feat_author.shape[1]
    n_cls = Wc.shape[1]
    n_t, n_c = feat_term.shape[0], feat_conf.shape[0]

    ft_pad = jnp.pad(feat_term, ((0, _BLK - n_t), (0, 0)))
    fc_pad = jnp.pad(feat_conf, ((_BLK - n_c, 0), (0, 0)))

    Wenc2 = jnp.transpose(Wenc, (0, 2, 1, 3)).reshape(_NLAYERS, _HID, _NMP * _HID)
    benc2 = benc.reshape(_NLAYERS, _NMP * _HID)
    WV = jnp.einsum('lmdk,lk->ldm', Wenc, Watt)
    WV = jnp.pad(WV, ((0, 0), (0, 0), (0, _HID - _NMP)))
    sb = jnp.einsum('lmk,lk->lm', benc, Watt) + batt[:, None]
    sb = jnp.pad(sb, ((0, 0), (0, _HID - _NMP)), constant_values=-1e9)
    ones_m = jnp.ones((_HID, _HID), jnp.float32)
    lane = jnp.arange(_NMP * _HID) // _HID
    sel = (lane[None, :] == jnp.arange(_HID)[:, None]).astype(jnp.float32)
    bc2 = bc.reshape(1, n_cls)
    bt3 = bt.reshape(4, 1, _HID)

    return pl.pallas_call(
        _fused_body,
        grid=(_NBLK,),
        in_specs=[
            pl.BlockSpec((_BLK, d_in), lambda i: (jnp.clip(i, 0, 1), 0)),
            pl.BlockSpec((_BLK, d_in), lambda i: (jnp.clip(i - _B0, 0, 1), 0)),
            pl.BlockSpec((_BLK, d_in), lambda i: (0, 0)),
            pl.BlockSpec((_BLK, d_in), lambda i: (0, 0)),
            pl.BlockSpec((4, d_in, _HID), lambda i: (0, 0, 0)),
            pl.BlockSpec((4, 1, _HID), lambda i: (0, 0, 0)),
            pl.BlockSpec((_NLAYERS, _HID, _NMP * _HID), lambda i: (0, 0, 0)),
            pl.BlockSpec((_NLAYERS, _NMP * _HID), lambda i: (0, 0)),
            pl.BlockSpec((_NLAYERS, _HID, _HID), lambda i: (0, 0, 0)),
            pl.BlockSpec((_NLAYERS, _HID), lambda i: (0, 0)),
            pl.BlockSpec((_HID, _HID), lambda i: (0, 0)),
            pl.BlockSpec((_HID, _NMP * _HID), lambda i: (0, 0)),
            pl.BlockSpec((_HID, n_cls), lambda i: (0, 0)),
            pl.BlockSpec((1, n_cls), lambda i: (0, 0)),
        ],
        out_specs=pl.BlockSpec((_BLK, n_cls), lambda i: (i, 0)),
        out_shape=jax.ShapeDtypeStruct((_NBLK * _BLK, n_cls), jnp.float32),
        compiler_params=pltpu.CompilerParams(
            dimension_semantics=("arbitrary",)),
    )(feat_author, feat_paper, ft_pad, fc_pad, Wt, bt3, Wenc2, benc2, WV, sb,
      ones_m, sel, Wc, bc2)


# speedup vs baseline: 1.2961x; 1.0153x over previous
import jax
import jax.numpy as jnp
from jax.experimental import pallas as pl
from jax.experimental.pallas import tpu as pltpu

_HID = 128
_NMP = 6
_NLAYERS = 2
_BLK = 2000
_B0, _B1, _B2 = 2, 4, 4
_NBLK = 5
_TERM_ROWS = 1500


def _fused_body(fa_ref, fp_ref, ft_ref, fc_ref, wt_ref, bt_ref, wenc_ref,
                benc_ref, wv_ref, sb_ref, ones_ref, sel_ref, wc_ref, bc_ref,
                out_ref):
    i = pl.program_id(0)
    t = jnp.where(i < _B0, 0, jnp.where(i < _B1, 1, 2))
    row = jax.lax.broadcasted_iota(jnp.int32, (_BLK, _HID), 0)
    primary = (i < _B2) | (row < _TERM_ROWS)
    ftc = jnp.concatenate([ft_ref[...], fc_ref[...]], axis=0)
    f = jnp.where(i < _B0, fa_ref[...],
                  jnp.where(i < _B1, fp_ref[...], ftc))
    h0 = jnp.dot(f, wt_ref[t], preferred_element_type=jnp.float32) + bt_ref[t, 0]
    h1 = jnp.dot(f, wt_ref[3], preferred_element_type=jnp.float32) + bt_ref[3, 0]
    h = jnp.where(primary, h0, h1)
    for l in range(_NLAYERS):
        s = jnp.dot(h, wv_ref[l],
                    preferred_element_type=jnp.float32) + sb_ref[l]
        s = jnp.where(s >= 0, s, 0.2 * s)
        e = jnp.exp(s)
        denom = jnp.dot(e, ones_ref[...],
                        preferred_element_type=jnp.float32)
        eb = jnp.dot(e, sel_ref[...],
                     preferred_element_type=jnp.float32)
        p = [eb[:, m * _HID:(m + 1) * _HID]
             * (jnp.dot(h, wenc_ref[l, m], preferred_element_type=jnp.float32)
                + benc_ref[l, m]) for m in range(_NMP)]
        acc = ((p[0] + p[1]) + (p[2] + p[3])) + (p[4] + p[5])
        acc = acc / denom
        h = jnp.where(acc > 0, acc, jnp.exp(jnp.minimum(acc, 0.0)) - 1.0)
    out_ref[...] = jnp.dot(h, wc_ref[...],
                           preferred_element_type=jnp.float32) + bc_ref[0]


def kernel(x, edge_index, feat_author, feat_paper, feat_term, feat_conf,
           Wt, bt, Wenc, benc, Watt, batt, Wc, bc):
    del x, edge_index
    d_in = feat_author.shape[1]
    n_cls = Wc.shape[1]
    n_t, n_c = feat_term.shape[0], feat_conf.shape[0]

    WV = jnp.einsum('lmdk,lk->ldm', Wenc, Watt)
    WV = jnp.pad(WV, ((0, 0), (0, 0), (0, _HID - _NMP)))
    sb = jnp.einsum('lmk,lk->lm', benc, Watt) + batt[:, None]
    sb = jnp.pad(sb, ((0, 0), (0, _HID - _NMP)), constant_values=-1e9)
    ones_m = jnp.ones((_HID, _HID), jnp.float32)
    lane = jnp.arange(_NMP * _HID) // _HID
    sel = (lane[None, :] == jnp.arange(_HID)[:, None]).astype(jnp.float32)
    bc2 = bc.reshape(1, n_cls)
    bt3 = bt.reshape(4, 1, _HID)

    return pl.pallas_call(
        _fused_body,
        grid=(_NBLK,),
        in_specs=[
            pl.BlockSpec((_BLK, d_in), lambda i: (jnp.clip(i, 0, 1), 0)),
            pl.BlockSpec((_BLK, d_in), lambda i: (jnp.clip(i - _B0, 0, 1), 0)),
            pl.BlockSpec((_TERM_ROWS, d_in), lambda i: (0, 0)),
            pl.BlockSpec((_BLK - _TERM_ROWS, d_in), lambda i: (0, 0)),
            pl.BlockSpec((4, d_in, _HID), lambda i: (0, 0, 0)),
            pl.BlockSpec((4, 1, _HID), lambda i: (0, 0, 0)),
            pl.BlockSpec((_NLAYERS, _NMP, _HID, _HID), lambda i: (0, 0, 0, 0)),
            pl.BlockSpec((_NLAYERS, _NMP, _HID), lambda i: (0, 0, 0)),
            pl.BlockSpec((_NLAYERS, _HID, _HID), lambda i: (0, 0, 0)),
            pl.BlockSpec((_NLAYERS, _HID), lambda i: (0, 0)),
            pl.BlockSpec((_HID, _HID), lambda i: (0, 0)),
            pl.BlockSpec((_HID, _NMP * _HID), lambda i: (0, 0)),
            pl.BlockSpec((_HID, n_cls), lambda i: (0, 0)),
            pl.BlockSpec((1, n_cls), lambda i: (0, 0)),
        ],
        out_specs=pl.BlockSpec((_BLK, n_cls), lambda i: (i, 0)),
        out_shape=jax.ShapeDtypeStruct((_NBLK * _BLK, n_cls), jnp.float32),
        compiler_params=pltpu.CompilerParams(
            dimension_semantics=("arbitrary",)),
    )(feat_author, feat_paper, feat_term, feat_conf, Wt, bt3, Wenc, benc, WV, sb,
      ones_m, sel, Wc, bc2)


# speedup vs baseline: 1.3383x; 1.0325x over previous
import jax
import jax.numpy as jnp
from jax.experimental import pallas as pl
from jax.experimental.pallas import tpu as pltpu

_HID = 128
_NMP = 6
_NLAYERS = 2
_BLK = 2000
_B0, _B1, _B2 = 2, 4, 4
_NBLK = 5
_TERM_ROWS = 1500


def _fused_body(fa_ref, fp_ref, ft_ref, fc_ref, wt_ref, bt_ref, wenc_ref,
                benc_ref, wv_ref, sb_ref, ones_ref, sel_ref, wc_ref, bc_ref,
                out_ref, wide_ref):
    i = pl.program_id(0)

    @pl.when(i == 0)
    def _():
        for l in range(_NLAYERS):
            for m in range(_NMP):
                wide_ref[l, :, m * _HID:(m + 1) * _HID] = wenc_ref[l, m]

    t = jnp.where(i < _B0, 0, jnp.where(i < _B1, 1, 2))
    row = jax.lax.broadcasted_iota(jnp.int32, (_BLK, _HID), 0)
    primary = (i < _B2) | (row < _TERM_ROWS)
    ftc = jnp.concatenate([ft_ref[...], fc_ref[...]], axis=0)
    f = jnp.where(i < _B0, fa_ref[...],
                  jnp.where(i < _B1, fp_ref[...], ftc))
    h0 = jnp.dot(f, wt_ref[t], preferred_element_type=jnp.float32) + bt_ref[t, 0]
    h1 = jnp.dot(f, wt_ref[3], preferred_element_type=jnp.float32) + bt_ref[3, 0]
    h = jnp.where(primary, h0, h1)
    for l in range(_NLAYERS):
        s = jnp.dot(h, wv_ref[l],
                    preferred_element_type=jnp.float32) + sb_ref[l]
        s = jnp.where(s >= 0, s, 0.2 * s)
        e = jnp.exp(s)
        denom = jnp.dot(e, ones_ref[...],
                        preferred_element_type=jnp.float32)
        eb = jnp.dot(e, sel_ref[...],
                     preferred_element_type=jnp.float32)
        outs = jnp.dot(h, wide_ref[l], preferred_element_type=jnp.float32)
        p = [eb[:, m * _HID:(m + 1) * _HID]
             * (outs[:, m * _HID:(m + 1) * _HID] + benc_ref[l, m])
             for m in range(_NMP)]
        acc = ((p[0] + p[1]) + (p[2] + p[3])) + (p[4] + p[5])
        acc = acc / denom
        h = jnp.where(acc > 0, acc, jnp.exp(jnp.minimum(acc, 0.0)) - 1.0)
    out_ref[...] = jnp.dot(h, wc_ref[...],
                           preferred_element_type=jnp.float32) + bc_ref[0]


def kernel(x, edge_index, feat_author, feat_paper, feat_term, feat_conf,
           Wt, bt, Wenc, benc, Watt, batt, Wc, bc):
    del x, edge_index
    d_in = feat_author.shape[1]
    n_cls = Wc.shape[1]
    n_t, n_c = feat_term.shape[0], feat_conf.shape[0]

    WV = jnp.einsum('lmdk,lk->ldm', Wenc, Watt)
    WV = jnp.pad(WV, ((0, 0), (0, 0), (0, _HID - _NMP)))
    sb = jnp.einsum('lmk,lk->lm', benc, Watt) + batt[:, None]
    sb = jnp.pad(sb, ((0, 0), (0, _HID - _NMP)), constant_values=-1e9)
    ones_m = jnp.ones((_HID, _HID), jnp.float32)
    lane = jnp.arange(_NMP * _HID) // _HID
    sel = (lane[None, :] == jnp.arange(_HID)[:, None]).astype(jnp.float32)
    bc2 = bc.reshape(1, n_cls)
    bt3 = bt.reshape(4, 1, _HID)

    return pl.pallas_call(
        _fused_body,
        grid=(_NBLK,),
        in_specs=[
            pl.BlockSpec((_BLK, d_in), lambda i: (jnp.clip(i, 0, 1), 0)),
            pl.BlockSpec((_BLK, d_in), lambda i: (jnp.clip(i - _B0, 0, 1), 0)),
            pl.BlockSpec((_TERM_ROWS, d_in), lambda i: (0, 0)),
            pl.BlockSpec((_BLK - _TERM_ROWS, d_in), lambda i: (0, 0)),
            pl.BlockSpec((4, d_in, _HID), lambda i: (0, 0, 0)),
            pl.BlockSpec((4, 1, _HID), lambda i: (0, 0, 0)),
            pl.BlockSpec((_NLAYERS, _NMP, _HID, _HID), lambda i: (0, 0, 0, 0)),
            pl.BlockSpec((_NLAYERS, _NMP, _HID), lambda i: (0, 0, 0)),
            pl.BlockSpec((_NLAYERS, _HID, _HID), lambda i: (0, 0, 0)),
            pl.BlockSpec((_NLAYERS, _HID), lambda i: (0, 0)),
            pl.BlockSpec((_HID, _HID), lambda i: (0, 0)),
            pl.BlockSpec((_HID, _NMP * _HID), lambda i: (0, 0)),
            pl.BlockSpec((_HID, n_cls), lambda i: (0, 0)),
            pl.BlockSpec((1, n_cls), lambda i: (0, 0)),
        ],
        out_specs=pl.BlockSpec((_BLK, n_cls), lambda i: (i, 0)),
        out_shape=jax.ShapeDtypeStruct((_NBLK * _BLK, n_cls), jnp.float32),
        compiler_params=pltpu.CompilerParams(
            dimension_semantics=("arbitrary",)),
        scratch_shapes=[pltpu.VMEM((_NLAYERS, _HID, _NMP * _HID), jnp.float32)],
    )(feat_author, feat_paper, feat_term, feat_conf, Wt, bt3, Wenc, benc, WV, sb,
      ones_m, sel, Wc, bc2)


# speedup vs baseline: 1.3437x; 1.0040x over previous
import jax
import jax.numpy as jnp
from jax.experimental import pallas as pl
from jax.experimental.pallas import tpu as pltpu

_HID = 128
_NMP = 6
_NLAYERS = 2
_BLK = 2000
_B0, _B1, _B2 = 2, 4, 4
_NBLK = 5
_TERM_ROWS = 1500


def _fused_body(fa_ref, fp_ref, ft_ref, fc_ref, wt_ref, bt_ref, wenc_ref,
                benc_ref, wv_ref, sb_ref, ones_ref, sel_ref, wc_ref, bc_ref,
                out_ref, wide_ref, h_ref):
    i = pl.program_id(0)

    @pl.when(i == 0)
    def _():
        for l in range(_NLAYERS):
            for m in range(_NMP):
                wide_ref[l, :, m * _HID:(m + 1) * _HID] = wenc_ref[l, m]

    t = jnp.where(i < _B0, 0, jnp.where(i < _B1, 1, 2))
    row = jax.lax.broadcasted_iota(jnp.int32, (_BLK, _HID), 0)
    primary = (i < _B2) | (row < _TERM_ROWS)
    ftc = jnp.concatenate([ft_ref[...], fc_ref[...]], axis=0)
    f = jnp.where(i < _B0, fa_ref[...],
                  jnp.where(i < _B1, fp_ref[...], ftc))
    h0 = jnp.dot(f, wt_ref[t], preferred_element_type=jnp.float32) + bt_ref[t, 0]
    h1 = jnp.dot(f, wt_ref[3], preferred_element_type=jnp.float32) + bt_ref[3, 0]
    h = jnp.where(primary, h0, h1)
    for l in range(_NLAYERS):
        s = jnp.dot(h, wv_ref[l],
                    preferred_element_type=jnp.float32) + sb_ref[l]
        s = jnp.where(s >= 0, s, 0.2 * s)
        e = jnp.exp(s)
        denom = jnp.dot(e, ones_ref[...],
                        preferred_element_type=jnp.float32)
        eb = jnp.dot(e, sel_ref[...],
                     preferred_element_type=jnp.float32)
        outs = jnp.dot(h, wide_ref[l], preferred_element_type=jnp.float32)
        p = [eb[:, m * _HID:(m + 1) * _HID]
             * (outs[:, m * _HID:(m + 1) * _HID] + benc_ref[l, m])
             for m in range(_NMP)]
        acc = ((p[0] + p[1]) + (p[2] + p[3])) + (p[4] + p[5])
        acc = acc / denom
        h = jnp.where(acc > 0, acc, jnp.exp(jnp.minimum(acc, 0.0)) - 1.0)
    out_ref[...] = jnp.dot(h, wc_ref[...],
                           preferred_element_type=jnp.float32) + bc_ref[0]


def kernel(x, edge_index, feat_author, feat_paper, feat_term, feat_conf,
           Wt, bt, Wenc, benc, Watt, batt, Wc, bc):
    del x, edge_index
    d_in = feat_author.shape[1]
    n_cls = Wc.shape[1]
    n_t, n_c = feat_term.shape[0], feat_conf.shape[0]

    WV = jnp.einsum('lmdk,lk->ldm', Wenc, Watt)
    WV = jnp.pad(WV, ((0, 0), (0, 0), (0, _HID - _NMP)))
    sb = jnp.einsum('lmk,lk->lm', benc, Watt) + batt[:, None]
    sb = jnp.pad(sb, ((0, 0), (0, _HID - _NMP)), constant_values=-1e9)
    ones_m = jnp.ones((_HID, _HID), jnp.float32)
    lane = jnp.arange(_NMP * _HID) // _HID
    sel = (lane[None, :] == jnp.arange(_HID)[:, None]).astype(jnp.float32)
    bc2 = bc.reshape(1, n_cls)
    bt3 = bt.reshape(4, 1, _HID)

    return pl.pallas_call(
        _fused_body,
        grid=(_NBLK,),
        in_specs=[
            pl.BlockSpec((_BLK, d_in), lambda i: (jnp.clip(i, 0, 1), 0)),
            pl.BlockSpec((_BLK, d_in), lambda i: (jnp.clip(i - _B0, 0, 1), 0)),
            pl.BlockSpec((_TERM_ROWS, d_in), lambda i: (0, 0)),
            pl.BlockSpec((_BLK - _TERM_ROWS, d_in), lambda i: (0, 0)),
            pl.BlockSpec((4, d_in, _HID), lambda i: (0, 0, 0)),
            pl.BlockSpec((4, 1, _HID), lambda i: (0, 0, 0)),
            pl.BlockSpec((_NLAYERS, _NMP, _HID, _HID), lambda i: (0, 0, 0, 0)),
            pl.BlockSpec((_NLAYERS, _NMP, _HID), lambda i: (0, 0, 0)),
            pl.BlockSpec((_NLAYERS, _HID, _HID), lambda i: (0, 0, 0)),
            pl.BlockSpec((_NLAYERS, _HID), lambda i: (0, 0)),
            pl.BlockSpec((_HID, _HID), lambda i: (0, 0)),
            pl.BlockSpec((_HID, _NMP * _HID), lambda i: (0, 0)),
            pl.BlockSpec((_HID, n_cls), lambda i: (0, 0)),
            pl.BlockSpec((1, n_cls), lambda i: (0, 0)),
        ],
        out_specs=pl.BlockSpec((_BLK, n_cls), lambda i: (i, 0)),
        out_shape=jax.ShapeDtypeStruct((_NBLK * _BLK, n_cls), jnp.float32),
        compiler_params=pltpu.CompilerParams(
            dimension_semantics=("arbitrary",)),
        scratch_shapes=[pltpu.VMEM((_NLAYERS, _HID, _NMP * _HID), jnp.float32),
                        pltpu.VMEM((_BLK, _HID), jnp.float32)],
    )(feat_author, feat_paper, feat_term, feat_conf, Wt, bt3, Wenc, benc, WV, sb,
      ones_m, sel, Wc, bc2)


# speedup vs baseline: 1.3554x; 1.0088x over previous
import jax
import jax.numpy as jnp
from jax.experimental import pallas as pl
from jax.experimental.pallas import tpu as pltpu

_HID = 128
_NMP = 6
_NLAYERS = 2
_BLK = 2000
_B0, _B1, _B2 = 2, 4, 4
_NBLK = 5
_TERM_ROWS = 1500


def _fused_body(fa_ref, fp_ref, ft_ref, fc_ref, wt_ref, bt_ref, wenc_ref,
                benc_ref, wvp_ref, ones_ref, sel_ref, wc_ref, bc_ref,
                out_ref, wide_ref, h_ref):
    i = pl.program_id(0)

    @pl.when(i == 0)
    def _():
        for l in range(_NLAYERS):
            for m in range(_NMP):
                wide_ref[l, :, m * _HID:(m + 1) * _HID] = wenc_ref[l, m]

    t = jnp.where(i < _B0, 0, jnp.where(i < _B1, 1, 2))
    row = jax.lax.broadcasted_iota(jnp.int32, (_BLK, _HID), 0)
    primary = (i < _B2) | (row < _TERM_ROWS)
    ftc = jnp.concatenate([ft_ref[...], fc_ref[...]], axis=0)
    f = jnp.where(i < _B0, fa_ref[...],
                  jnp.where(i < _B1, fp_ref[...], ftc))
    h0 = jnp.dot(f, wt_ref[t], preferred_element_type=jnp.float32) + bt_ref[t, 0]
    h1 = jnp.dot(f, wt_ref[3], preferred_element_type=jnp.float32) + bt_ref[3, 0]
    h = jnp.where(primary, h0, h1)
    for l in range(_NLAYERS):
        s = (jnp.dot(h, wvp_ref[l, 0:_HID, :],
                     preferred_element_type=jnp.float32)
             + wvp_ref[l, _HID:_HID + 8, :][0:1])
        s = jnp.where(s >= 0, s, 0.2 * s)
        e = jnp.exp(s)
        denom = jnp.dot(e, ones_ref[...],
                        preferred_element_type=jnp.float32)
        eb = jnp.dot(e, sel_ref[...],
                     preferred_element_type=jnp.float32)
        outs = jnp.dot(h, wide_ref[l], preferred_element_type=jnp.float32)
        p = [eb[:, m * _HID:(m + 1) * _HID]
             * (outs[:, m * _HID:(m + 1) * _HID] + benc_ref[l, m])
             for m in range(_NMP)]
        acc = ((p[0] + p[1]) + (p[2] + p[3])) + (p[4] + p[5])
        acc = acc / denom
        h = jnp.where(acc > 0, acc, jnp.exp(jnp.minimum(acc, 0.0)) - 1.0)
    out_ref[...] = jnp.dot(h, wc_ref[...],
                           preferred_element_type=jnp.float32) + bc_ref[0]


def kernel(x, edge_index, feat_author, feat_paper, feat_term, feat_conf,
           Wt, bt, Wenc, benc, Watt, batt, Wc, bc):
    del x, edge_index
    d_in = feat_author.shape[1]
    n_cls = Wc.shape[1]
    n_t, n_c = feat_term.shape[0], feat_conf.shape[0]

    WV = jnp.einsum('lmdk,lk->ldm', Wenc, Watt)
    WV = jnp.pad(WV, ((0, 0), (0, 0), (0, _HID - _NMP)))
    sb = jnp.einsum('lmk,lk->lm', benc, Watt) + batt[:, None]
    sb = jnp.pad(sb, ((0, 0), (0, _HID - _NMP)), constant_values=-1e9)
    WVP = jnp.concatenate(
        [WV, sb[:, None, :], jnp.zeros((_NLAYERS, 7, _HID), jnp.float32)], axis=1)
    ones_m = jnp.ones((_HID, _HID), jnp.float32)
    lane = jnp.arange(_NMP * _HID) // _HID
    sel = (lane[None, :] == jnp.arange(_HID)[:, None]).astype(jnp.float32)
    bc2 = bc.reshape(1, n_cls)
    bt3 = bt.reshape(4, 1, _HID)

    return pl.pallas_call(
        _fused_body,
        grid=(_NBLK,),
        in_specs=[
            pl.BlockSpec((_BLK, d_in), lambda i: (jnp.clip(i, 0, 1), 0)),
            pl.BlockSpec((_BLK, d_in), lambda i: (jnp.clip(i - _B0, 0, 1), 0)),
            pl.BlockSpec((_TERM_ROWS, d_in), lambda i: (0, 0)),
            pl.BlockSpec((_BLK - _TERM_ROWS, d_in), lambda i: (0, 0)),
            pl.BlockSpec((4, d_in, _HID), lambda i: (0, 0, 0)),
            pl.BlockSpec((4, 1, _HID), lambda i: (0, 0, 0)),
            pl.BlockSpec((_NLAYERS, _NMP, _HID, _HID), lambda i: (0, 0, 0, 0)),
            pl.BlockSpec((_NLAYERS, _NMP, _HID), lambda i: (0, 0, 0)),
            pl.BlockSpec((_NLAYERS, _HID + 8, _HID), lambda i: (0, 0, 0)),
            pl.BlockSpec((_HID, _HID), lambda i: (0, 0)),
            pl.BlockSpec((_HID, _NMP * _HID), lambda i: (0, 0)),
            pl.BlockSpec((_HID, n_cls), lambda i: (0, 0)),
            pl.BlockSpec((1, n_cls), lambda i: (0, 0)),
        ],
        out_specs=pl.BlockSpec((_BLK, n_cls), lambda i: (i, 0)),
        out_shape=jax.ShapeDtypeStruct((_NBLK * _BLK, n_cls), jnp.float32),
        compiler_params=pltpu.CompilerParams(
            dimension_semantics=("arbitrary",)),
        scratch_shapes=[pltpu.VMEM((_NLAYERS, _HID, _NMP * _HID), jnp.float32),
                        pltpu.VMEM((_BLK, _HID), jnp.float32)],
    )(feat_author, feat_paper, feat_term, feat_conf, Wt, bt3, Wenc, benc, WVP,
      ones_m, sel, Wc, bc2)


# speedup vs baseline: 1.4356x; 1.0591x over previous
import jax
import jax.numpy as jnp
from jax.experimental import pallas as pl
from jax.experimental.pallas import tpu as pltpu

_HID = 128
_NMP = 6
_NLAYERS = 2
_BLK = 2000
_B0, _B1, _B2 = 2, 4, 4
_NBLK = 5
_TERM_ROWS = 1500


def _fused_body(fa_ref, fp_ref, ft_ref, fc_ref, wt_ref, bt_ref, wenc_ref,
                benc_ref, watt_ref, batt_ref, ones_ref, sel_ref, wc_ref,
                bc_ref, out_ref, wide_ref, h_ref, wvp_ref):
    i = pl.program_id(0)

    @pl.when(i == 0)
    def _():
        wvp_ref[...] = jnp.full((_NLAYERS, _HID + 8, _HID), -1e9, jnp.float32)
        for l in range(_NLAYERS):
            watt_col = jnp.swapaxes(watt_ref[l:l + 1, :], 0, 1)
            for m in range(_NMP):
                wide_ref[l, :, m * _HID:(m + 1) * _HID] = wenc_ref[l, m]
                wvp_ref[l, 0:_HID, m:m + 1] = jnp.dot(
                    wenc_ref[l, m], watt_col,
                    preferred_element_type=jnp.float32)
            wvp_ref[l, 0:_HID, _NMP:_HID] = jnp.zeros((_HID, _HID - _NMP),
                                                      jnp.float32)
            sbv = jnp.dot(benc_ref[l], watt_col,
                          preferred_element_type=jnp.float32)
            wvp_ref[l, _HID:_HID + 1, 0:_NMP] = (
                jnp.swapaxes(sbv, 0, 1) + batt_ref[l:l + 1, 0:1])

    t = jnp.where(i < _B0, 0, jnp.where(i < _B1, 1, 2))
    row = jax.lax.broadcasted_iota(jnp.int32, (_BLK, _HID), 0)
    primary = (i < _B2) | (row < _TERM_ROWS)
    ftc = jnp.concatenate([ft_ref[...], fc_ref[...]], axis=0)
    f = jnp.where(i < _B0, fa_ref[...],
                  jnp.where(i < _B1, fp_ref[...], ftc))
    h0 = jnp.dot(f, wt_ref[t], preferred_element_type=jnp.float32) + bt_ref[t, 0]
    h1 = jnp.dot(f, wt_ref[3], preferred_element_type=jnp.float32) + bt_ref[3, 0]
    h = jnp.where(primary, h0, h1)
    for l in range(_NLAYERS):
        s = (jnp.dot(h, wvp_ref[l, 0:_HID, :],
                     preferred_element_type=jnp.float32)
             + wvp_ref[l, _HID:_HID + 8, :][0:1])
        s = jnp.where(s >= 0, s, 0.2 * s)
        e = jnp.exp(s)
        denom = jnp.dot(e, ones_ref[...],
                        preferred_element_type=jnp.float32)
        eb = jnp.dot(e, sel_ref[...],
                     preferred_element_type=jnp.float32)
        outs = jnp.dot(h, wide_ref[l], preferred_element_type=jnp.float32)
        p = [eb[:, m * _HID:(m + 1) * _HID]
             * (outs[:, m * _HID:(m + 1) * _HID] + benc_ref[l, m])
             for m in range(_NMP)]
        acc = ((p[0] + p[1]) + (p[2] + p[3])) + (p[4] + p[5])
        acc = acc / denom
        h = jnp.where(acc > 0, acc, jnp.exp(jnp.minimum(acc, 0.0)) - 1.0)
    out_ref[...] = jnp.dot(h, wc_ref[...],
                           preferred_element_type=jnp.float32) + bc_ref[0]


def kernel(x, edge_index, feat_author, feat_paper, feat_term, feat_conf,
           Wt, bt, Wenc, benc, Watt, batt, Wc, bc):
    del x, edge_index
    d_in = feat_author.shape[1]
    n_cls = Wc.shape[1]
    n_t, n_c = feat_term.shape[0], feat_conf.shape[0]

    batt2 = batt.reshape(_NLAYERS, 1)
    ones_m = jnp.ones((_HID, _HID), jnp.float32)
    lane = jnp.arange(_NMP * _HID) // _HID
    sel = (lane[None, :] == jnp.arange(_HID)[:, None]).astype(jnp.float32)
    bc2 = bc.reshape(1, n_cls)
    bt3 = bt.reshape(4, 1, _HID)

    return pl.pallas_call(
        _fused_body,
        grid=(_NBLK,),
        in_specs=[
            pl.BlockSpec((_BLK, d_in), lambda i: (jnp.clip(i, 0, 1), 0)),
            pl.BlockSpec((_BLK, d_in), lambda i: (jnp.clip(i - _B0, 0, 1), 0)),
            pl.BlockSpec((_TERM_ROWS, d_in), lambda i: (0, 0)),
            pl.BlockSpec((_BLK - _TERM_ROWS, d_in), lambda i: (0, 0)),
            pl.BlockSpec((4, d_in, _HID), lambda i: (0, 0, 0)),
            pl.BlockSpec((4, 1, _HID), lambda i: (0, 0, 0)),
            pl.BlockSpec((_NLAYERS, _NMP, _HID, _HID), lambda i: (0, 0, 0, 0)),
            pl.BlockSpec((_NLAYERS, _NMP, _HID), lambda i: (0, 0, 0)),
            pl.BlockSpec((_NLAYERS, _HID), lambda i: (0, 0)),
            pl.BlockSpec((_NLAYERS, 1), lambda i: (0, 0)),
            pl.BlockSpec((_HID, _HID), lambda i: (0, 0)),
            pl.BlockSpec((_HID, _NMP * _HID), lambda i: (0, 0)),
            pl.BlockSpec((_HID, n_cls), lambda i: (0, 0)),
            pl.BlockSpec((1, n_cls), lambda i: (0, 0)),
        ],
        out_specs=pl.BlockSpec((_BLK, n_cls), lambda i: (i, 0)),
        out_shape=jax.ShapeDtypeStruct((_NBLK * _BLK, n_cls), jnp.float32),
        compiler_params=pltpu.CompilerParams(
            dimension_semantics=("arbitrary",)),
        scratch_shapes=[pltpu.VMEM((_NLAYERS, _HID, _NMP * _HID), jnp.float32),
                        pltpu.VMEM((_BLK, _HID), jnp.float32),
                        pltpu.VMEM((_NLAYERS, _HID + 8, _HID), jnp.float32)],
    )(feat_author, feat_paper, feat_term, feat_conf, Wt, bt3, Wenc, benc, Watt,
      batt2, ones_m, sel, Wc, bc2)


# speedup vs baseline: 1.4360x; 1.0003x over previous
import jax
import jax.numpy as jnp
from jax.experimental import pallas as pl
from jax.experimental.pallas import tpu as pltpu

_HID = 128
_NMP = 6
_NLAYERS = 2
_BLK = 2000
_B0, _B1, _B2 = 2, 4, 4
_NBLK = 5
_TERM_ROWS = 1500


def _fused_body(fa_ref, fp_ref, ft_ref, fc_ref, wt_ref, bt_ref, wenc_ref,
                benc_ref, watt_ref, batt_ref, ones_ref, sel_ref, wc_ref,
                bc_ref, out_ref, wide_ref, wvp_ref):
    i = pl.program_id(0)

    @pl.when(i == 0)
    def _():
        wvp_ref[...] = jnp.full((_NLAYERS, _HID + 8, _HID), -1e9, jnp.float32)
        for l in range(_NLAYERS):
            watt_col = jnp.swapaxes(watt_ref[l:l + 1, :], 0, 1)
            for m in range(_NMP):
                wide_ref[l, :, m * _HID:(m + 1) * _HID] = wenc_ref[l, m]
                wvp_ref[l, 0:_HID, m:m + 1] = jnp.dot(
                    wenc_ref[l, m], watt_col,
                    preferred_element_type=jnp.float32)
            wvp_ref[l, 0:_HID, _NMP:_HID] = jnp.zeros((_HID, _HID - _NMP),
                                                      jnp.float32)
            sbv = jnp.dot(benc_ref[l], watt_col,
                          preferred_element_type=jnp.float32)
            wvp_ref[l, _HID:_HID + 1, 0:_NMP] = (
                jnp.swapaxes(sbv, 0, 1) + batt_ref[l:l + 1, 0:1])

    t = jnp.where(i < _B0, 0, jnp.where(i < _B1, 1, 2))
    row = jax.lax.broadcasted_iota(jnp.int32, (_BLK, _HID), 0)
    primary = (i < _B2) | (row < _TERM_ROWS)
    ftc = jnp.concatenate([ft_ref[...], fc_ref[...]], axis=0)
    f = jnp.where(i < _B0, fa_ref[...],
                  jnp.where(i < _B1, fp_ref[...], ftc))
    h0 = jnp.dot(f, wt_ref[t], preferred_element_type=jnp.float32) + bt_ref[t, 0]
    h1 = jnp.dot(f, wt_ref[3], preferred_element_type=jnp.float32) + bt_ref[3, 0]
    h = jnp.where(primary, h0, h1)
    for l in range(_NLAYERS):
        s = (jnp.dot(h, wvp_ref[l, 0:_HID, :],
                     preferred_element_type=jnp.float32)
             + wvp_ref[l, _HID:_HID + 8, :][0:1])
        s = jnp.where(s >= 0, s, 0.2 * s)
        e = jnp.exp(s)
        denom = jnp.dot(e, ones_ref[...],
                        preferred_element_type=jnp.float32)
        eb = jnp.dot(e, sel_ref[...],
                     preferred_element_type=jnp.float32)
        outs = jnp.dot(h, wide_ref[l], preferred_element_type=jnp.float32)
        p = [eb[:, m * _HID:(m + 1) * _HID]
             * (outs[:, m * _HID:(m + 1) * _HID] + benc_ref[l, m])
             for m in range(_NMP)]
        acc = ((p[0] + p[1]) + (p[2] + p[3])) + (p[4] + p[5])
        acc = acc / denom
        h = jnp.where(acc > 0, acc, jnp.exp(jnp.minimum(acc, 0.0)) - 1.0)
    out_ref[...] = jnp.dot(h, wc_ref[...],
                           preferred_element_type=jnp.float32) + bc_ref[0]


def kernel(x, edge_index, feat_author, feat_paper, feat_term, feat_conf,
           Wt, bt, Wenc, benc, Watt, batt, Wc, bc):
    del x, edge_index
    d_in = feat_author.shape[1]
    n_cls = Wc.shape[1]
    n_t, n_c = feat_term.shape[0], feat_conf.shape[0]

    batt2 = batt.reshape(_NLAYERS, 1)
    ones_m = jnp.ones((_HID, _HID), jnp.float32)
    lane = jnp.arange(_NMP * _HID) // _HID
    sel = (lane[None, :] == jnp.arange(_HID)[:, None]).astype(jnp.float32)
    bc2 = bc.reshape(1, n_cls)
    bt3 = bt.reshape(4, 1, _HID)

    return pl.pallas_call(
        _fused_body,
        grid=(_NBLK,),
        in_specs=[
            pl.BlockSpec((_BLK, d_in), lambda i: (jnp.clip(i, 0, 1), 0)),
            pl.BlockSpec((_BLK, d_in), lambda i: (jnp.clip(i - _B0, 0, 1), 0)),
            pl.BlockSpec((_TERM_ROWS, d_in), lambda i: (0, 0)),
            pl.BlockSpec((_BLK - _TERM_ROWS, d_in), lambda i: (0, 0)),
            pl.BlockSpec((4, d_in, _HID), lambda i: (0, 0, 0)),
            pl.BlockSpec((4, 1, _HID), lambda i: (0, 0, 0)),
            pl.BlockSpec((_NLAYERS, _NMP, _HID, _HID), lambda i: (0, 0, 0, 0)),
            pl.BlockSpec((_NLAYERS, _NMP, _HID), lambda i: (0, 0, 0)),
            pl.BlockSpec((_NLAYERS, _HID), lambda i: (0, 0)),
            pl.BlockSpec((_NLAYERS, 1), lambda i: (0, 0)),
            pl.BlockSpec((_HID, _HID), lambda i: (0, 0)),
            pl.BlockSpec((_HID, _NMP * _HID), lambda i: (0, 0)),
            pl.BlockSpec((_HID, n_cls), lambda i: (0, 0)),
            pl.BlockSpec((1, n_cls), lambda i: (0, 0)),
        ],
        out_specs=pl.BlockSpec((_BLK, n_cls), lambda i: (i, 0)),
        out_shape=jax.ShapeDtypeStruct((_NBLK * _BLK, n_cls), jnp.float32),
        compiler_params=pltpu.CompilerParams(
            dimension_semantics=("arbitrary",)),
        scratch_shapes=[pltpu.VMEM((_NLAYERS, _HID, _NMP * _HID), jnp.float32),
                        pltpu.VMEM((_NLAYERS, _HID + 8, _HID), jnp.float32)],
    )(feat_author, feat_paper, feat_term, feat_conf, Wt, bt3, Wenc, benc, Watt,
      batt2, ones_m, sel, Wc, bc2)
